# baseline (speedup 1.0000x reference)
import jax
import jax.numpy as jnp
import numpy as np
from jax import lax
from jax.experimental import pallas as pl
from jax.experimental.pallas import tpu as pltpu
from jax.experimental.pallas import tpu_sc as plsc

B = 128
N = 8192
K = 32
L = 16
NC = 2
NS = 16
NW = NC * NS
R = B // NW
NV = N // L
CSTRIDE = N + L

INT_MIN = np.int32(-2147483648)
MASK31 = np.int32(0x7FFFFFFF)


def _digit(ub, shift):
    return lax.shift_right_logical(ub, np.int32(shift)) & np.int32(0xF)


def _ub_of(v):
    bb = lax.bitcast_convert_type(v, jnp.int32)
    key = bb ^ (lax.shift_right_arithmetic(bb, 31) & MASK31)
    return key ^ INT_MIN


def _sc_body(x_hbm, out_hbm, xv, c0, c1, hist):
    wid = lax.axis_index("s") * NC + lax.axis_index("c")
    lane = lax.iota(jnp.int32, L)
    lane16 = lane * L
    ones = jnp.ones((L,), jnp.int32)
    zeros16 = jnp.zeros((L,), jnp.int32)

    base = wid * (R * N)
    pltpu.sync_copy(x_hbm.at[pl.ds(base, R * N)], xv)

    for t in range(R * 256 // L):
        hist[pl.ds(t * L, L)] = zeros16

    def l0_body(i, carry):
        for j in range(R):
            v = xv[pl.ds(j * N + i * L, L)]
            d = _digit(_ub_of(v), 28)
            plsc.addupdate_scatter(
                hist, [np.int32(j * 256) + lane16 + d], ones)
        return carry

    lax.fori_loop(0, NV, l0_body, 0)

    def pick_bucket(j, kr):
        hb = j * 256
        comb = hist[pl.ds(hb, L)]
        for l in range(1, L):
            comb = comb + hist[pl.ds(hb + l * L, L)]
        suf = jnp.flip(plsc.cumsum(jnp.flip(comb)))
        ge = suf >= kr
        bsel = jnp.sum(ge.astype(jnp.int32)) - 1
        c_above = jnp.sum(jnp.where(ge, 0, comb))
        mnew = jnp.sum(jnp.where(lane == bsel, comb, 0))
        return bsel, kr - c_above, mnew

    bsels, krs, ms = [], [], []
    for j in range(R):
        bsel, kr, m = pick_bucket(j, np.int32(K))
        bsels.append(bsel)
        krs.append(kr)
        ms.append(m)
    tkeys = [lax.shift_left(bsels[j], 28) for j in range(R)]

    def compact0_body(i, offs):
        new = []
        for j in range(R):
            v = xv[pl.ds(j * N + i * L, L)]
            ub = _ub_of(v)
            msk = _digit(ub, 28) == bsels[j]
            plsc.store_compressed(
                c0.at[pl.ds(j * CSTRIDE + offs[j], L)], ub, mask=msk)
            pc = plsc.all_reduce_population_count(msk)
            new.append(offs[j] + pc[0])
        return tuple(new)

    lax.fori_loop(0, NV, compact0_body, (np.int32(0),) * R)

    tvecs = []
    for j in range(R):
        kr, m, tkey = krs[j], ms[j], tkeys[j]
        src, dst = c0, c1
        for shift in range(24, -1, -4):
            for l in range(L):
                hist[pl.ds(j * 256 + l * L, L)] = zeros16
            nv = lax.shift_right_logical(m + np.int32(L - 1), np.int32(4))

            def hist_body(i, carry, src=src, m=m, shift=shift, j=j):
                ub = src[pl.ds(j * CSTRIDE + i * L, L)]
                valid = lane < (m - i * L)
                d = _digit(ub, shift)
                plsc.addupdate_scatter(
                    hist, [np.int32(j * 256) + lane16 + d], ones, mask=valid)
                return carry

            lax.fori_loop(0, nv, hist_body, 0)
            bsel, kr, m2 = pick_bucket(j, kr)
            tkey = tkey | lax.shift_left(bsel, shift)

            if shift > 0:
                def compact_body(i, off, src=src, dst=dst, m=m, shift=shift,
                                 bsel=bsel, j=j):
                    ub = src[pl.ds(j * CSTRIDE + i * L, L)]
                    valid = lane < (m - i * L)
                    msk = valid & (_digit(ub, shift) == bsel)
                    plsc.store_compressed(
                        dst.at[pl.ds(j * CSTRIDE + off, L)], ub, mask=msk)
                    pc = plsc.all_reduce_population_count(msk)
                    return off + pc[0]

                lax.fori_loop(0, nv, compact_body, np.int32(0))
                src, dst = dst, src
            m = m2

        key_t = tkey ^ INT_MIN
        fb = key_t ^ (lax.shift_right_arithmetic(key_t, 31) & MASK31)
        tvecs.append(
            lax.bitcast_convert_type(jnp.full((L,), fb, jnp.int32),
                                     jnp.float32))

    def mask_body(i, carry):
        for j in range(R):
            sl = pl.ds(j * N + i * L, L)
            v = xv[sl]
            xv[sl] = jnp.where(v >= tvecs[j], v, np.float32(0.0))
        return carry

    lax.fori_loop(0, NV, mask_body, 0)
    pltpu.sync_copy(xv, out_hbm.at[pl.ds(base, R * N)])


@jax.jit
def kernel(x):
    mesh = plsc.VectorSubcoreMesh(
        core_axis_name="c", subcore_axis_name="s", num_cores=NC,
        num_subcores=NS)
    out = pl.kernel(
        _sc_body,
        out_type=jax.ShapeDtypeStruct((B * N,), jnp.float32),
        mesh=mesh,
        compiler_params=pltpu.CompilerParams(needs_layout_passes=False),
        scratch_types=[
            pltpu.VMEM((R * N,), jnp.float32),
            pltpu.VMEM((R * CSTRIDE,), jnp.int32),
            pltpu.VMEM((R * CSTRIDE,), jnp.int32),
            pltpu.VMEM((R * 256,), jnp.int32),
        ],
    )(x.reshape(-1))
    return out.reshape(B, N)


# speedup vs baseline: 1.7687x; 1.7687x over previous
import jax
import jax.numpy as jnp
import numpy as np
from jax import lax
from jax.experimental import pallas as pl
from jax.experimental.pallas import tpu as pltpu
from jax.experimental.pallas import tpu_sc as plsc

B = 128
N = 8192
K = 32
L = 16
NC = 2
NS = 16
NW = NC * NS
R = B // NW
NV = N // L
CSTRIDE = N + L

INT_MIN = np.int32(-2147483648)
MASK31 = np.int32(0x7FFFFFFF)


def _digit(ub, shift):
    return lax.shift_right_logical(ub, np.int32(shift)) & np.int32(0xF)


def _ub_of(v):
    bb = lax.bitcast_convert_type(v, jnp.int32)
    return bb ^ (lax.shift_right_arithmetic(bb, 31) | INT_MIN)


def _sc_body(x_hbm, out_hbm, xv, c0, c1, hist):
    wid = lax.axis_index("s") * NC + lax.axis_index("c")
    lane = lax.iota(jnp.int32, L)
    lane16 = lane * L
    ones = jnp.ones((L,), jnp.int32)
    zeros16 = jnp.zeros((L,), jnp.int32)

    base = wid * (R * N)
    pltpu.sync_copy(x_hbm.at[pl.ds(base, R * N)], xv)

    for t in range(R * 256 // L):
        hist[pl.ds(t * L, L)] = zeros16

    @plsc.parallel_loop(0, NV, unroll=4)
    def l0_body(i):
        for j in range(R):
            v = xv[pl.ds(j * N + i * L, L)]
            d = _digit(_ub_of(v), 28)
            plsc.addupdate_scatter(
                hist, [np.int32(j * 256) + lane16 + d], ones)

    def pick_bucket(j, kr):
        hb = j * 256
        comb = hist[pl.ds(hb, L)]
        for l in range(1, L):
            comb = comb + hist[pl.ds(hb + l * L, L)]
        suf = jnp.flip(plsc.cumsum(jnp.flip(comb)))
        ge = suf >= kr
        bsel = jnp.sum(ge.astype(jnp.int32)) - 1
        c_above = jnp.sum(jnp.where(ge, 0, comb))
        mnew = jnp.sum(jnp.where(lane == bsel, comb, 0))
        return bsel, kr - c_above, mnew

    bsels, krs, ms = [], [], []
    for j in range(R):
        bsel, kr, m = pick_bucket(j, np.int32(K))
        bsels.append(bsel)
        krs.append(kr)
        ms.append(m)
    tkeys = [lax.shift_left(bsels[j], 28) for j in range(R)]

    @plsc.parallel_loop(0, NV, unroll=2, carry=(jnp.int32(0),) * R)
    def compact0_body(i, offs):
        new = []
        for j in range(R):
            v = xv[pl.ds(j * N + i * L, L)]
            ub = _ub_of(v)
            msk = _digit(ub, 28) == bsels[j]
            plsc.store_compressed(
                c0.at[pl.ds(j * CSTRIDE + offs[j], L)], ub, mask=msk)
            pc = plsc.all_reduce_population_count(msk)
            new.append(offs[j] + pc[0])
        return tuple(new)

    tvecs = []
    for j in range(R):
        kr, m, tkey = krs[j], ms[j], tkeys[j]
        src, dst = c0, c1
        for shift in range(24, -1, -4):
            for l in range(L):
                hist[pl.ds(j * 256 + l * L, L)] = zeros16
            nv = lax.shift_right_logical(m + np.int32(L - 1), np.int32(4))

            def hist_body(i, carry, src=src, m=m, shift=shift, j=j):
                ub = src[pl.ds(j * CSTRIDE + i * L, L)]
                valid = lane < (m - i * L)
                d = _digit(ub, shift)
                plsc.addupdate_scatter(
                    hist, [np.int32(j * 256) + lane16 + d], ones, mask=valid)
                return carry

            lax.fori_loop(0, nv, hist_body, 0)
            bsel, kr, m2 = pick_bucket(j, kr)
            tkey = tkey | lax.shift_left(bsel, shift)

            if shift > 0:
                def compact_body(i, off, src=src, dst=dst, m=m, shift=shift,
                                 bsel=bsel, j=j):
                    ub = src[pl.ds(j * CSTRIDE + i * L, L)]
                    valid = lane < (m - i * L)
                    msk = valid & (_digit(ub, shift) == bsel)
                    plsc.store_compressed(
                        dst.at[pl.ds(j * CSTRIDE + off, L)], ub, mask=msk)
                    pc = plsc.all_reduce_population_count(msk)
                    return off + pc[0]

                lax.fori_loop(0, nv, compact_body, np.int32(0))
                src, dst = dst, src
            m = m2

        key_t = tkey ^ INT_MIN
        fb = key_t ^ (lax.shift_right_arithmetic(key_t, 31) & MASK31)
        tvecs.append(
            lax.bitcast_convert_type(jnp.full((L,), fb, jnp.int32),
                                     jnp.float32))

    @plsc.parallel_loop(0, NV, unroll=4)
    def mask_body(i):
        for j in range(R):
            sl = pl.ds(j * N + i * L, L)
            v = xv[sl]
            xv[sl] = jnp.where(v >= tvecs[j], v, np.float32(0.0))
    pltpu.sync_copy(xv, out_hbm.at[pl.ds(base, R * N)])


@jax.jit
def kernel(x):
    mesh = plsc.VectorSubcoreMesh(
        core_axis_name="c", subcore_axis_name="s", num_cores=NC,
        num_subcores=NS)
    out = pl.kernel(
        _sc_body,
        out_type=jax.ShapeDtypeStruct((B * N,), jnp.float32),
        mesh=mesh,
        compiler_params=pltpu.CompilerParams(needs_layout_passes=False),
        scratch_types=[
            pltpu.VMEM((R * N,), jnp.float32),
            pltpu.VMEM((R * CSTRIDE,), jnp.int32),
            pltpu.VMEM((R * CSTRIDE,), jnp.int32),
            pltpu.VMEM((R * 256,), jnp.int32),
        ],
    )(x.reshape(-1))
    return out.reshape(B, N)


# speedup vs baseline: 1.9791x; 1.1189x over previous
import jax
import jax.numpy as jnp
import numpy as np
from jax import lax
from jax.experimental import pallas as pl
from jax.experimental.pallas import tpu as pltpu
from jax.experimental.pallas import tpu_sc as plsc

B = 128
N = 8192
K = 32
L = 16
NC = 2
NS = 16
NW = NC * NS
R = B // NW
NV = N // L
CSTRIDE = N + L

INT_MIN = np.int32(-2147483648)
MASK31 = np.int32(0x7FFFFFFF)


def _digit(ub, shift):
    return lax.shift_right_logical(ub, np.int32(shift)) & np.int32(0xF)


def _ub_of(v):
    bb = lax.bitcast_convert_type(v, jnp.int32)
    return bb ^ (lax.shift_right_arithmetic(bb, 31) | INT_MIN)


def _sc_body(x_hbm, out_hbm, xv, c0, c1, hist):
    wid = lax.axis_index("s") * NC + lax.axis_index("c")
    lane = lax.iota(jnp.int32, L)
    lane16 = lane * L
    ones = jnp.ones((L,), jnp.int32)
    zeros16 = jnp.zeros((L,), jnp.int32)

    row0 = wid * R
    pltpu.sync_copy(x_hbm.at[pl.ds(row0, R)], xv)

    for t in range(R * 256 // L):
        hist[pl.ds(t * L, L)] = zeros16

    @plsc.parallel_loop(0, NV, unroll=4)
    def l0_body(i):
        for j in range(R):
            v = xv[j, pl.ds(i * L, L)]
            d = _digit(_ub_of(v), 28)
            plsc.addupdate_scatter(
                hist, [np.int32(j * 256) + lane16 + d], ones)

    def pick_bucket(j, kr):
        hb = j * 256
        comb = hist[pl.ds(hb, L)]
        for l in range(1, L):
            comb = comb + hist[pl.ds(hb + l * L, L)]
        suf = jnp.flip(plsc.cumsum(jnp.flip(comb)))
        ge = suf >= kr
        bsel = jnp.sum(ge.astype(jnp.int32)) - 1
        c_above = jnp.sum(jnp.where(ge, 0, comb))
        mnew = jnp.sum(jnp.where(lane == bsel, comb, 0))
        return bsel, kr - c_above, mnew

    bsels, krs, ms = [], [], []
    for j in range(R):
        bsel, kr, m = pick_bucket(j, np.int32(K))
        bsels.append(bsel)
        krs.append(kr)
        ms.append(m)
    tkeys = [lax.shift_left(bsels[j], 28) for j in range(R)]

    @plsc.parallel_loop(0, NV, unroll=2, carry=(jnp.int32(0),) * R)
    def compact0_body(i, offs):
        new = []
        for j in range(R):
            v = xv[j, pl.ds(i * L, L)]
            ub = _ub_of(v)
            msk = _digit(ub, 28) == bsels[j]
            plsc.store_compressed(
                c0.at[pl.ds(j * CSTRIDE + offs[j], L)], ub, mask=msk)
            pc = plsc.all_reduce_population_count(msk)
            new.append(offs[j] + pc[0])
        return tuple(new)

    tvecs = []
    for j in range(R):
        kr, m, tkey = krs[j], ms[j], tkeys[j]
        src, dst = c0, c1
        for shift in range(24, -1, -4):
            for l in range(L):
                hist[pl.ds(j * 256 + l * L, L)] = zeros16
            nv = lax.shift_right_logical(m + np.int32(L - 1), np.int32(4))

            def hist_body(i, carry, src=src, m=m, shift=shift, j=j):
                ub = src[pl.ds(j * CSTRIDE + i * L, L)]
                valid = lane < (m - i * L)
                d = _digit(ub, shift)
                plsc.addupdate_scatter(
                    hist, [np.int32(j * 256) + lane16 + d], ones, mask=valid)
                return carry

            lax.fori_loop(0, nv, hist_body, 0)
            bsel, kr, m2 = pick_bucket(j, kr)
            tkey = tkey | lax.shift_left(bsel, shift)

            if shift > 0:
                def compact_body(i, off, src=src, dst=dst, m=m, shift=shift,
                                 bsel=bsel, j=j):
                    ub = src[pl.ds(j * CSTRIDE + i * L, L)]
                    valid = lane < (m - i * L)
                    msk = valid & (_digit(ub, shift) == bsel)
                    plsc.store_compressed(
                        dst.at[pl.ds(j * CSTRIDE + off, L)], ub, mask=msk)
                    pc = plsc.all_reduce_population_count(msk)
                    return off + pc[0]

                lax.fori_loop(0, nv, compact_body, np.int32(0))
                src, dst = dst, src
            m = m2

        key_t = tkey ^ INT_MIN
        fb = key_t ^ (lax.shift_right_arithmetic(key_t, 31) & MASK31)
        tvecs.append(
            lax.bitcast_convert_type(jnp.full((L,), fb, jnp.int32),
                                     jnp.float32))

    @plsc.parallel_loop(0, NV, unroll=4)
    def mask_body(i):
        for j in range(R):
            sl = pl.ds(i * L, L)
            v = xv[j, sl]
            xv[j, sl] = jnp.where(v >= tvecs[j], v, np.float32(0.0))
    pltpu.sync_copy(xv, out_hbm.at[pl.ds(row0, R)])


@jax.jit
def kernel(x):
    mesh = plsc.VectorSubcoreMesh(
        core_axis_name="c", subcore_axis_name="s", num_cores=NC,
        num_subcores=NS)
    out = pl.kernel(
        _sc_body,
        out_type=jax.ShapeDtypeStruct((B, N), jnp.float32),
        mesh=mesh,
        compiler_params=pltpu.CompilerParams(
            needs_layout_passes=False, use_tc_tiling_on_sc=True),
        scratch_types=[
            pltpu.VMEM((R, N), jnp.float32),
            pltpu.VMEM((R * CSTRIDE,), jnp.int32),
            pltpu.VMEM((R * CSTRIDE,), jnp.int32),
            pltpu.VMEM((R * 256,), jnp.int32),
        ],
    )(x)
    return out


# speedup vs baseline: 2.1185x; 1.0704x over previous
import jax
import jax.numpy as jnp
import numpy as np
from jax import lax
from jax.experimental import pallas as pl
from jax.experimental.pallas import tpu as pltpu
from jax.experimental.pallas import tpu_sc as plsc

B = 128
N = 8192
K = 32
L = 16
NC = 2
NS = 16
NW = NC * NS
R = B // NW
NV = N // L
CSTRIDE = N + L

INT_MIN = np.int32(-2147483648)
MASK31 = np.int32(0x7FFFFFFF)


def _digit(ub, shift):
    return lax.shift_right_logical(ub, np.int32(shift)) & np.int32(0xF)


def _ub_of(v):
    bb = lax.bitcast_convert_type(v, jnp.int32)
    return bb ^ (lax.shift_right_arithmetic(bb, 31) | INT_MIN)


def _sc_body(x_hbm, out_hbm, xv, c0, c1, hist):
    wid = lax.axis_index("s") * NC + lax.axis_index("c")
    lane = lax.iota(jnp.int32, L)
    lane16 = lane * L
    ones = jnp.ones((L,), jnp.int32)
    zeros16 = jnp.zeros((L,), jnp.int32)

    row0 = wid * R
    pltpu.sync_copy(x_hbm.at[pl.ds(row0, R)], xv)

    for t in range(R * 256 // L):
        hist[pl.ds(t * L, L)] = zeros16

    @plsc.parallel_loop(0, NV, unroll=4)
    def l0_body(i):
        for j in range(R):
            v = xv[j, pl.ds(i * L, L)]
            d = lax.shift_right_logical(_ub_of(v), np.int32(28))
            plsc.addupdate_scatter(
                hist, [np.int32(j * 256) + lane16 + d], ones)

    def pick(suf, kr):
        ge = suf >= kr
        bsel = plsc.all_reduce_population_count(ge) - np.int32(1)
        s_at = suf.at[bsel].get(mode="promise_in_bounds")
        nxt = jnp.minimum(bsel + np.int32(1), np.int32(L - 1))
        s_next = jnp.where(bsel == np.int32(L - 1), np.int32(0),
                           suf.at[nxt].get(mode="promise_in_bounds"))
        return bsel, kr - s_next, s_at - s_next

    def suffix(comb):
        return jnp.flip(plsc.cumsum(jnp.flip(comb)))

    bsels, krs, ms, tkeys = [], [], [], []
    for j in range(R):
        comb = hist[pl.ds(j * 256, L)]
        for l in range(1, L):
            comb = comb + hist[pl.ds(j * 256 + l * L, L)]
        b, kr, m = pick(suffix(comb), jnp.full((L,), K, jnp.int32))
        bsels.append(b)
        krs.append(kr)
        ms.append(m)
        tkeys.append(jnp.left_shift(b, np.int32(28)))

    @plsc.parallel_loop(0, NV, unroll=2, carry=(jnp.int32(0),) * R)
    def compact0_body(i, offs):
        new = []
        for j in range(R):
            v = xv[j, pl.ds(i * L, L)]
            ub = _ub_of(v)
            msk = lax.shift_right_logical(ub, np.int32(28)) == bsels[j]
            plsc.store_compressed(
                c0.at[pl.ds(j * CSTRIDE + offs[j], L)], ub, mask=msk)
            pc = plsc.all_reduce_population_count(msk)
            new.append(offs[j] + pc[0])
        return tuple(new)

    tvecs = []
    for j in range(R):
        kr, m, tkey = krs[j], ms[j], tkeys[j]
        src, dst = c0, c1
        hb = np.int32(j * 256)
        for shift in range(24, -1, -4):
            hist[pl.ds(j * 256, L)] = zeros16
            nv = lax.shift_right_logical(m[0] + np.int32(L - 1), np.int32(4))

            def hist_body(i, carry, src=src, m=m, shift=shift, j=j, hb=hb):
                ub = src[pl.ds(j * CSTRIDE + i * L, L)]
                valid = (lane + i * L) < m
                d = _digit(ub, shift)
                plsc.addupdate_scatter(hist, [hb + d], ones, mask=valid)
                return carry

            lax.fori_loop(0, nv, hist_body, 0)
            bsel, kr, m2 = pick(suffix(hist[pl.ds(j * 256, L)]), kr)
            tkey = tkey | jnp.left_shift(bsel, np.int32(shift))

            if shift > 0:
                def compact_body(i, off, src=src, dst=dst, m=m, shift=shift,
                                 bsel=bsel, j=j):
                    ub = src[pl.ds(j * CSTRIDE + i * L, L)]
                    valid = (lane + i * L) < m
                    msk = valid & (_digit(ub, shift) == bsel)
                    plsc.store_compressed(
                        dst.at[pl.ds(j * CSTRIDE + off, L)], ub, mask=msk)
                    pc = plsc.all_reduce_population_count(msk)
                    return off + pc[0]

                lax.fori_loop(0, nv, compact_body, jnp.int32(0))
                src, dst = dst, src
            m = m2

        key_t = tkey ^ INT_MIN
        fb = key_t ^ (lax.shift_right_arithmetic(key_t, 31) & MASK31)
        tvecs.append(lax.bitcast_convert_type(fb, jnp.float32))

    @plsc.parallel_loop(0, NV, unroll=4)
    def mask_body(i):
        for j in range(R):
            sl = pl.ds(i * L, L)
            v = xv[j, sl]
            xv[j, sl] = jnp.where(v >= tvecs[j], v, np.float32(0.0))
    pltpu.sync_copy(xv, out_hbm.at[pl.ds(row0, R)])


@jax.jit
def kernel(x):
    mesh = plsc.VectorSubcoreMesh(
        core_axis_name="c", subcore_axis_name="s", num_cores=NC,
        num_subcores=NS)
    out = pl.kernel(
        _sc_body,
        out_type=jax.ShapeDtypeStruct((B, N), jnp.float32),
        mesh=mesh,
        compiler_params=pltpu.CompilerParams(
            needs_layout_passes=False, use_tc_tiling_on_sc=True),
        scratch_types=[
            pltpu.VMEM((R, N), jnp.float32),
            pltpu.VMEM((R * CSTRIDE,), jnp.int32),
            pltpu.VMEM((R * CSTRIDE,), jnp.int32),
            pltpu.VMEM((R * 256,), jnp.int32),
        ],
    )(x)
    return out


# speedup vs baseline: 2.2687x; 1.0709x over previous
import jax
import jax.numpy as jnp
import numpy as np
from jax import lax
from jax.experimental import pallas as pl
from jax.experimental.pallas import tpu as pltpu
from jax.experimental.pallas import tpu_sc as plsc

B = 128
N = 8192
K = 32
L = 16
NC = 2
NS = 16
NW = NC * NS
B_SC = 64
R = B_SC // NW
TB = 16
NV = N // L
CSTRIDE = N + L

INT_MIN = np.int32(-2147483648)
MASK31 = np.int32(0x7FFFFFFF)


def _digit(ub, shift):
    return lax.shift_right_logical(ub, np.int32(shift)) & np.int32(0xF)


def _ub_of(v):
    bb = lax.bitcast_convert_type(v, jnp.int32)
    return bb ^ (lax.shift_right_arithmetic(bb, 31) | INT_MIN)


def _sc_body(x_hbm, out_hbm, xv, c0, c1, hist):
    wid = lax.axis_index("s") * NC + lax.axis_index("c")
    lane = lax.iota(jnp.int32, L)
    lane16 = lane * L
    ones = jnp.ones((L,), jnp.int32)
    zeros16 = jnp.zeros((L,), jnp.int32)

    row0 = wid * R
    pltpu.sync_copy(x_hbm.at[pl.ds(row0, R)], xv)

    for t in range(R * 256 // L):
        hist[pl.ds(t * L, L)] = zeros16

    @plsc.parallel_loop(0, NV, unroll=4)
    def l0_body(i):
        for j in range(R):
            v = xv[j, pl.ds(i * L, L)]
            d = lax.shift_right_logical(_ub_of(v), np.int32(28))
            plsc.addupdate_scatter(
                hist, [np.int32(j * 256) + lane16 + d], ones)

    def pick(suf, kr):
        ge = suf >= kr
        bsel = plsc.all_reduce_population_count(ge) - np.int32(1)
        s_at = suf.at[bsel].get(mode="promise_in_bounds")
        nxt = jnp.minimum(bsel + np.int32(1), np.int32(L - 1))
        s_next = jnp.where(bsel == np.int32(L - 1), np.int32(0),
                           suf.at[nxt].get(mode="promise_in_bounds"))
        return bsel, kr - s_next, s_at - s_next

    def suffix(comb):
        return jnp.flip(plsc.cumsum(jnp.flip(comb)))

    bsels, krs, ms, tkeys = [], [], [], []
    for j in range(R):
        comb = hist[pl.ds(j * 256, L)]
        for l in range(1, L):
            comb = comb + hist[pl.ds(j * 256 + l * L, L)]
        b, kr, m = pick(suffix(comb), jnp.full((L,), K, jnp.int32))
        bsels.append(b)
        krs.append(kr)
        ms.append(m)
        tkeys.append(jnp.left_shift(b, np.int32(28)))

    @plsc.parallel_loop(0, NV, unroll=2, carry=(jnp.int32(0),) * R)
    def compact0_body(i, offs):
        new = []
        for j in range(R):
            v = xv[j, pl.ds(i * L, L)]
            ub = _ub_of(v)
            msk = lax.shift_right_logical(ub, np.int32(28)) == bsels[j]
            plsc.store_compressed(
                c0.at[pl.ds(j * CSTRIDE + offs[j], L)], ub, mask=msk)
            pc = plsc.all_reduce_population_count(msk)
            new.append(offs[j] + pc[0])
        return tuple(new)

    tvecs = []
    for j in range(R):
        kr, m, tkey = krs[j], ms[j], tkeys[j]
        src, dst = c0, c1
        hb = np.int32(j * 256)
        for shift in range(24, -1, -4):
            hist[pl.ds(j * 256, L)] = zeros16
            nv = lax.shift_right_logical(m[0] + np.int32(L - 1), np.int32(4))

            def hist_body(i, carry, src=src, m=m, shift=shift, j=j, hb=hb):
                ub = src[pl.ds(j * CSTRIDE + i * L, L)]
                valid = (lane + i * L) < m
                d = _digit(ub, shift)
                plsc.addupdate_scatter(hist, [hb + d], ones, mask=valid)
                return carry

            lax.fori_loop(0, nv, hist_body, 0)
            bsel, kr, m2 = pick(suffix(hist[pl.ds(j * 256, L)]), kr)
            tkey = tkey | jnp.left_shift(bsel, np.int32(shift))

            if shift > 0:
                def compact_body(i, off, src=src, dst=dst, m=m, shift=shift,
                                 bsel=bsel, j=j):
                    ub = src[pl.ds(j * CSTRIDE + i * L, L)]
                    valid = (lane + i * L) < m
                    msk = valid & (_digit(ub, shift) == bsel)
                    plsc.store_compressed(
                        dst.at[pl.ds(j * CSTRIDE + off, L)], ub, mask=msk)
                    pc = plsc.all_reduce_population_count(msk)
                    return off + pc[0]

                lax.fori_loop(0, nv, compact_body, jnp.int32(0))
                src, dst = dst, src
            m = m2

        key_t = tkey ^ INT_MIN
        fb = key_t ^ (lax.shift_right_arithmetic(key_t, 31) & MASK31)
        tvecs.append(lax.bitcast_convert_type(fb, jnp.float32))

    @plsc.parallel_loop(0, NV, unroll=4)
    def mask_body(i):
        for j in range(R):
            sl = pl.ds(i * L, L)
            v = xv[j, sl]
            xv[j, sl] = jnp.where(v >= tvecs[j], v, np.float32(0.0))
    pltpu.sync_copy(xv, out_hbm.at[pl.ds(row0, R)])


def _tc_body(x_ref, o_ref):
    x = x_ref[...]
    b = lax.bitcast_convert_type(x, jnp.int32)
    key = b ^ (lax.shift_right_arithmetic(b, 31) & MASK31)

    def it(i, off):
        cand = off | jnp.left_shift(np.int32(1), np.int32(31) - i)
        thr = INT_MIN + cand
        cnt = jnp.sum((key >= thr).astype(jnp.int32), axis=1, keepdims=True)
        return jnp.where(cnt >= K, cand, off)

    off = lax.fori_loop(0, 32, it, jnp.zeros((TB, 1), jnp.int32))
    tkey = INT_MIN + off
    fb = tkey ^ (lax.shift_right_arithmetic(tkey, 31) & MASK31)
    t = lax.bitcast_convert_type(fb, jnp.float32)
    o_ref[...] = jnp.where(x >= t, x, np.float32(0.0))


@jax.jit
def kernel(x):
    mesh = plsc.VectorSubcoreMesh(
        core_axis_name="c", subcore_axis_name="s", num_cores=NC,
        num_subcores=NS)
    out = pl.kernel(
        _sc_body,
        out_type=jax.ShapeDtypeStruct((B, N), jnp.float32),
        mesh=mesh,
        compiler_params=pltpu.CompilerParams(
            needs_layout_passes=False, use_tc_tiling_on_sc=True),
        scratch_types=[
            pltpu.VMEM((R, N), jnp.float32),
            pltpu.VMEM((R * CSTRIDE,), jnp.int32),
            pltpu.VMEM((R * CSTRIDE,), jnp.int32),
            pltpu.VMEM((R * 256,), jnp.int32),
        ],
    )(x)
    n_tc_blocks = (B - B_SC) // TB
    out_tc = pl.pallas_call(
        _tc_body,
        grid=(n_tc_blocks,),
        in_specs=[pl.BlockSpec((TB, N), lambda i: (i + B_SC // TB, 0))],
        out_specs=pl.BlockSpec((TB, N), lambda i: (i, 0)),
        out_shape=jax.ShapeDtypeStruct((B - B_SC, N), jnp.float32),
    )(x)
    return lax.dynamic_update_slice(out, out_tc, (B_SC, 0))


# speedup vs baseline: 2.4016x; 1.0585x over previous
import jax
import jax.numpy as jnp
import numpy as np
from jax import lax
from jax.experimental import pallas as pl
from jax.experimental.pallas import tpu as pltpu
from jax.experimental.pallas import tpu_sc as plsc

B = 128
N = 8192
K = 32
L = 16
NC = 2
NS = 16
NW = NC * NS
B_SC = 96
R = B_SC // NW
TB = 8
NV = N // L
CSTRIDE = N + L

INT_MIN = np.int32(-2147483648)
MASK31 = np.int32(0x7FFFFFFF)


def _digit(ub, shift):
    return lax.shift_right_logical(ub, np.int32(shift)) & np.int32(0xF)


def _ub_of(v):
    bb = lax.bitcast_convert_type(v, jnp.int32)
    return bb ^ (lax.shift_right_arithmetic(bb, 31) | INT_MIN)


def _sc_body(x_hbm, out_hbm, xv, c0, c1, hist):
    wid = lax.axis_index("s") * NC + lax.axis_index("c")
    lane = lax.iota(jnp.int32, L)
    lane16 = lane * L
    ones = jnp.ones((L,), jnp.int32)
    zeros16 = jnp.zeros((L,), jnp.int32)

    for j in range(R):
        pltpu.sync_copy(x_hbm.at[pl.ds(wid + j * NW, 1)],
                        xv.at[pl.ds(j, 1)])

    for t in range(R * 256 // L):
        hist[pl.ds(t * L, L)] = zeros16

    @plsc.parallel_loop(0, NV, unroll=4)
    def l0_body(i):
        for j in range(R):
            v = xv[j, pl.ds(i * L, L)]
            d = lax.shift_right_logical(_ub_of(v), np.int32(28))
            plsc.addupdate_scatter(
                hist, [np.int32(j * 256) + lane16 + d], ones)

    def pick(suf, kr):
        ge = suf >= kr
        bsel = plsc.all_reduce_population_count(ge) - np.int32(1)
        s_at = suf.at[bsel].get(mode="promise_in_bounds")
        nxt = jnp.minimum(bsel + np.int32(1), np.int32(L - 1))
        s_next = jnp.where(bsel == np.int32(L - 1), np.int32(0),
                           suf.at[nxt].get(mode="promise_in_bounds"))
        return bsel, kr - s_next, s_at - s_next

    def suffix(comb):
        return jnp.flip(plsc.cumsum(jnp.flip(comb)))

    bsels, krs, ms, tkeys = [], [], [], []
    for j in range(R):
        comb = hist[pl.ds(j * 256, L)]
        for l in range(1, L):
            comb = comb + hist[pl.ds(j * 256 + l * L, L)]
        b, kr, m = pick(suffix(comb), jnp.full((L,), K, jnp.int32))
        bsels.append(b)
        krs.append(kr)
        ms.append(m)
        tkeys.append(jnp.left_shift(b, np.int32(28)))

    for j in range(R):
        hist[pl.ds(j * 256, L)] = zeros16

    @plsc.parallel_loop(0, NV, unroll=2, carry=(jnp.int32(0),) * R)
    def compact0_body(i, offs):
        new = []
        for j in range(R):
            v = xv[j, pl.ds(i * L, L)]
            ub = _ub_of(v)
            msk = lax.shift_right_logical(ub, np.int32(28)) == bsels[j]
            plsc.store_compressed(
                c0.at[pl.ds(j * CSTRIDE + offs[j], L)], ub, mask=msk)
            plsc.addupdate_scatter(
                hist, [np.int32(j * 256) + _digit(ub, 24)], ones, mask=msk)
            pc = plsc.all_reduce_population_count(msk)
            new.append(offs[j] + pc[0])
        return tuple(new)

    tvecs = []
    for j in range(R):
        kr, m, tkey = krs[j], ms[j], tkeys[j]
        src, dst = c0, c1
        hb = np.int32(j * 256)
        for shift in range(24, -1, -4):
            bsel, kr, m2 = pick(suffix(hist[pl.ds(j * 256, L)]), kr)
            tkey = tkey | jnp.left_shift(bsel, np.int32(shift))

            if shift > 0:
                hist[pl.ds(j * 256, L)] = zeros16
                nv = lax.shift_right_logical(
                    m[0] + np.int32(L - 1), np.int32(4))

                def compact_body(i, off, src=src, dst=dst, m=m, shift=shift,
                                 bsel=bsel, j=j, hb=hb):
                    ub = src[pl.ds(j * CSTRIDE + i * L, L)]
                    valid = (lane + i * L) < m
                    msk = valid & (_digit(ub, shift) == bsel)
                    plsc.store_compressed(
                        dst.at[pl.ds(j * CSTRIDE + off, L)], ub, mask=msk)
                    plsc.addupdate_scatter(
                        hist, [hb + _digit(ub, shift - 4)], ones, mask=msk)
                    pc = plsc.all_reduce_population_count(msk)
                    return off + pc[0]

                lax.fori_loop(0, nv, compact_body, jnp.int32(0))
                src, dst = dst, src
            m = m2

        key_t = tkey ^ INT_MIN
        fb = key_t ^ (lax.shift_right_arithmetic(key_t, 31) & MASK31)
        tvecs.append(lax.bitcast_convert_type(fb, jnp.float32))

    @plsc.parallel_loop(0, NV, unroll=4)
    def mask_body(i):
        for j in range(R):
            sl = pl.ds(i * L, L)
            v = xv[j, sl]
            xv[j, sl] = jnp.where(v >= tvecs[j], v, np.float32(0.0))
    for j in range(R):
        pltpu.sync_copy(xv.at[pl.ds(j, 1)],
                        out_hbm.at[pl.ds(wid + j * NW, 1)])


def _tc_body(x_ref, o_ref):
    x = x_ref[...]
    b = lax.bitcast_convert_type(x, jnp.int32)
    key = b ^ (lax.shift_right_arithmetic(b, 31) & MASK31)

    def it(i, off):
        cand = off | jnp.left_shift(np.int32(1), np.int32(31) - i)
        thr = INT_MIN + cand
        cnt = jnp.sum((key >= thr).astype(jnp.int32), axis=1, keepdims=True)
        return jnp.where(cnt >= K, cand, off)

    off = lax.fori_loop(0, 32, it, jnp.zeros((TB, 1), jnp.int32))
    tkey = INT_MIN + off
    fb = tkey ^ (lax.shift_right_arithmetic(tkey, 31) & MASK31)
    t = lax.bitcast_convert_type(fb, jnp.float32)
    o_ref[...] = jnp.where(x >= t, x, np.float32(0.0))


@jax.jit
def kernel(x):
    mesh = plsc.VectorSubcoreMesh(
        core_axis_name="c", subcore_axis_name="s", num_cores=NC,
        num_subcores=NS)
    out = pl.kernel(
        _sc_body,
        out_type=jax.ShapeDtypeStruct((B, N), jnp.float32),
        mesh=mesh,
        compiler_params=pltpu.CompilerParams(
            needs_layout_passes=False, use_tc_tiling_on_sc=True),
        scratch_types=[
            pltpu.VMEM((R, N), jnp.float32),
            pltpu.VMEM((R * CSTRIDE,), jnp.int32),
            pltpu.VMEM((R * CSTRIDE,), jnp.int32),
            pltpu.VMEM((R * 256,), jnp.int32),
        ],
    )(x)
    n_tc_blocks = (B - B_SC) // TB
    out_tc = pl.pallas_call(
        _tc_body,
        grid=(n_tc_blocks,),
        in_specs=[pl.BlockSpec((TB, N), lambda i: (i + B_SC // TB, 0))],
        out_specs=pl.BlockSpec((TB, N), lambda i: (i, 0)),
        out_shape=jax.ShapeDtypeStruct((B - B_SC, N), jnp.float32),
    )(x)
    return lax.dynamic_update_slice(out, out_tc, (B_SC, 0))


# speedup vs baseline: 2.5756x; 1.0725x over previous
import jax
import jax.numpy as jnp
import numpy as np
from jax import lax
from jax.experimental import pallas as pl
from jax.experimental.pallas import tpu as pltpu
from jax.experimental.pallas import tpu_sc as plsc

B = 128
N = 8192
K = 32
L = 16
NC = 2
NS = 16
NW = NC * NS
B_SC = 64
R = B_SC // NW
TB = 32
NV = N // L
CSTRIDE = N + L

INT_MIN = np.int32(-2147483648)
MASK31 = np.int32(0x7FFFFFFF)


def _digit(ub, shift):
    return lax.shift_right_logical(ub, np.int32(shift)) & np.int32(0xF)


def _ub_of(v):
    bb = lax.bitcast_convert_type(v, jnp.int32)
    return bb ^ (lax.shift_right_arithmetic(bb, 31) | INT_MIN)


def _sc_body(x_hbm, out_hbm, xv, c0, c1, hist):
    wid = lax.axis_index("s") * NC + lax.axis_index("c")
    lane = lax.iota(jnp.int32, L)
    lane16 = lane * L
    ones = jnp.ones((L,), jnp.int32)
    zeros16 = jnp.zeros((L,), jnp.int32)

    for j in range(R):
        pltpu.sync_copy(x_hbm.at[pl.ds(wid + j * NW, 1)],
                        xv.at[pl.ds(j, 1)])

    for t in range(R * 256 // L):
        hist[pl.ds(t * L, L)] = zeros16

    @plsc.parallel_loop(0, NV, unroll=4)
    def l0_body(i):
        for j in range(R):
            v = xv[j, pl.ds(i * L, L)]
            d = lax.shift_right_logical(_ub_of(v), np.int32(28))
            plsc.addupdate_scatter(
                hist, [np.int32(j * 256) + lane16 + d], ones)

    def pick(suf, kr):
        ge = suf >= kr
        bsel = plsc.all_reduce_population_count(ge) - np.int32(1)
        s_at = suf.at[bsel].get(mode="promise_in_bounds")
        nxt = jnp.minimum(bsel + np.int32(1), np.int32(L - 1))
        s_next = jnp.where(bsel == np.int32(L - 1), np.int32(0),
                           suf.at[nxt].get(mode="promise_in_bounds"))
        return bsel, kr - s_next, s_at - s_next

    def suffix(comb):
        return jnp.flip(plsc.cumsum(jnp.flip(comb)))

    bsels, krs, ms, tkeys = [], [], [], []
    for j in range(R):
        comb = hist[pl.ds(j * 256, L)]
        for l in range(1, L):
            comb = comb + hist[pl.ds(j * 256 + l * L, L)]
        b, kr, m = pick(suffix(comb), jnp.full((L,), K, jnp.int32))
        bsels.append(b)
        krs.append(kr)
        ms.append(m)
        tkeys.append(jnp.left_shift(b, np.int32(28)))

    for j in range(R):
        hist[pl.ds(j * 256, L)] = zeros16

    @plsc.parallel_loop(0, NV, unroll=2, carry=(jnp.int32(0),) * R)
    def compact0_body(i, offs):
        new = []
        for j in range(R):
            v = xv[j, pl.ds(i * L, L)]
            ub = _ub_of(v)
            msk = lax.shift_right_logical(ub, np.int32(28)) == bsels[j]
            plsc.store_compressed(
                c0.at[pl.ds(j * CSTRIDE + offs[j], L)], ub, mask=msk)
            plsc.addupdate_scatter(
                hist, [np.int32(j * 256) + _digit(ub, 24)], ones, mask=msk)
            pc = plsc.all_reduce_population_count(msk)
            new.append(offs[j] + pc[0])
        return tuple(new)

    tvecs = []
    for j in range(R):
        kr, m, tkey = krs[j], ms[j], tkeys[j]
        src, dst = c0, c1
        hb = np.int32(j * 256)
        for shift in range(24, -1, -4):
            bsel, kr, m2 = pick(suffix(hist[pl.ds(j * 256, L)]), kr)
            tkey = tkey | jnp.left_shift(bsel, np.int32(shift))

            if shift > 0:
                hist[pl.ds(j * 256, L)] = zeros16
                nv = lax.shift_right_logical(
                    m[0] + np.int32(L - 1), np.int32(4))

                def compact_body(i, off, src=src, dst=dst, m=m, shift=shift,
                                 bsel=bsel, j=j, hb=hb):
                    ub = src[pl.ds(j * CSTRIDE + i * L, L)]
                    valid = (lane + i * L) < m
                    msk = valid & (_digit(ub, shift) == bsel)
                    plsc.store_compressed(
                        dst.at[pl.ds(j * CSTRIDE + off, L)], ub, mask=msk)
                    plsc.addupdate_scatter(
                        hist, [hb + _digit(ub, shift - 4)], ones, mask=msk)
                    pc = plsc.all_reduce_population_count(msk)
                    return off + pc[0]

                lax.fori_loop(0, nv, compact_body, jnp.int32(0))
                src, dst = dst, src
            m = m2

        key_t = tkey ^ INT_MIN
        fb = key_t ^ (lax.shift_right_arithmetic(key_t, 31) & MASK31)
        tvecs.append(lax.bitcast_convert_type(fb, jnp.float32))

    @plsc.parallel_loop(0, NV, unroll=4)
    def mask_body(i):
        for j in range(R):
            sl = pl.ds(i * L, L)
            v = xv[j, sl]
            xv[j, sl] = jnp.where(v >= tvecs[j], v, np.float32(0.0))
    for j in range(R):
        pltpu.sync_copy(xv.at[pl.ds(j, 1)],
                        out_hbm.at[pl.ds(wid + j * NW, 1)])


def _tc_body(x_ref, o_ref):
    x = x_ref[...]
    b = lax.bitcast_convert_type(x, jnp.int32)
    key = b ^ (lax.shift_right_arithmetic(b, 31) & MASK31)

    def it(i, off):
        cand = off | jnp.left_shift(np.int32(1), np.int32(31) - i)
        thr = INT_MIN + cand
        cnt = jnp.sum((key >= thr).astype(jnp.int32), axis=1, keepdims=True)
        return jnp.where(cnt >= K, cand, off)

    off = lax.fori_loop(0, 32, it, jnp.zeros((TB, 1), jnp.int32))
    tkey = INT_MIN + off
    fb = tkey ^ (lax.shift_right_arithmetic(tkey, 31) & MASK31)
    t = lax.bitcast_convert_type(fb, jnp.float32)
    o_ref[...] = jnp.where(x >= t, x, np.float32(0.0))


@jax.jit
def kernel(x):
    mesh = plsc.VectorSubcoreMesh(
        core_axis_name="c", subcore_axis_name="s", num_cores=NC,
        num_subcores=NS)
    out = pl.kernel(
        _sc_body,
        out_type=jax.ShapeDtypeStruct((B, N), jnp.float32),
        mesh=mesh,
        compiler_params=pltpu.CompilerParams(
            needs_layout_passes=False, use_tc_tiling_on_sc=True),
        scratch_types=[
            pltpu.VMEM((R, N), jnp.float32),
            pltpu.VMEM((R * CSTRIDE,), jnp.int32),
            pltpu.VMEM((R * CSTRIDE,), jnp.int32),
            pltpu.VMEM((R * 256,), jnp.int32),
        ],
    )(x)
    n_tc_blocks = (B - B_SC) // TB
    out_tc = pl.pallas_call(
        _tc_body,
        grid=(n_tc_blocks,),
        in_specs=[pl.BlockSpec((TB, N), lambda i: (i + B_SC // TB, 0))],
        out_specs=pl.BlockSpec((TB, N), lambda i: (i, 0)),
        out_shape=jax.ShapeDtypeStruct((B - B_SC, N), jnp.float32),
    )(x)
    return lax.dynamic_update_slice(out, out_tc, (B_SC, 0))
